# baseline (speedup 1.0000x reference)
import jax
import jax.numpy as jnp
from jax import lax
from jax.experimental import pallas as pl
from jax.experimental.pallas import tpu as pltpu
from jax.experimental.pallas import tpu_sc as plsc

N = 10000
E = 320000
C_IN = 128
C_AUG = 144
HID = 256
NC, NS = 2, 16
NPAD = 10240
ROWS_PER_TILE = NPAD // NS
CHUNK = 100
NCHUNK_A = E // (NC * NS) // CHUNK
NCHUNK_B = E // NS // CHUNK
CPS_A = 20
CHUNK_B = 128
NCHUNK_B2 = 20480 // CHUNK_B
CPS_B = 40
R = 400
G = N // R

_mesh = plsc.VectorSubcoreMesh(core_axis_name="c", subcore_axis_name="s")


def _pipelined_segs(gather_src, srcs_slice, dsts_slice, nseg, cps, table,
                    src_v, dst_v, rows0, rows1, g0, g1, gather_cols=None):
    nrows = rows0.shape[0]

    def gwin(buf):
        if gather_cols is None:
            return buf
        return buf.at[pl.ds(0, nrows), pl.ds(0, gather_cols)]

    def seg(g, carry):
        pltpu.sync_copy(srcs_slice(g), src_v)
        pltpu.sync_copy(dsts_slice(g), dst_v)

        def chunk(j, c):
            pltpu.async_copy(gather_src.at[src_v.at[j]], gwin(rows0), g0)
            pltpu.make_async_copy(gather_src.at[src_v.at[j]], gwin(rows0), g0).wait()
            pltpu.sync_copy(rows0, table.at[dst_v.at[j]], add=True)
            return c

        return lax.fori_loop(0, cps, chunk, carry)

    lax.fori_loop(0, nseg, seg, 0)


def _agg1_body(xaug, srcs, dsts, zeros, out, src_v, dst_v, rows0, rows1,
               table, g0, g1):
    cid = lax.axis_index("c")
    sid = lax.axis_index("s")
    pltpu.sync_copy(zeros, table.at[pl.ds(sid * ROWS_PER_TILE, ROWS_PER_TILE)])
    plsc.subcore_barrier()
    _pipelined_segs(
        xaug,
        lambda g: srcs.at[cid, sid, pl.ds(g * CPS_A, CPS_A)],
        lambda g: dsts.at[cid, sid, pl.ds(g * CPS_A, CPS_A)],
        NCHUNK_A // CPS_A, CPS_A, table, src_v, dst_v, rows0, rows1, g0, g1)
    plsc.subcore_barrier()
    sl = pl.ds(sid * ROWS_PER_TILE, ROWS_PER_TILE)
    pltpu.sync_copy(table.at[sl], out.at[cid, sl])


_agg1 = pl.kernel(
    _agg1_body,
    out_type=jax.ShapeDtypeStruct((NC, NPAD, C_AUG), jnp.float32),
    mesh=_mesh,
    compiler_params=pltpu.CompilerParams(use_tc_tiling_on_sc=False),
    scratch_types=[
        pltpu.VMEM((CPS_A, CHUNK), jnp.int32),
        pltpu.VMEM((CPS_A, CHUNK), jnp.int32),
        pltpu.VMEM((CHUNK, C_AUG), jnp.float32),
        pltpu.VMEM((CHUNK, C_AUG), jnp.float32),
        pltpu.VMEM_SHARED((NPAD, C_AUG), jnp.float32),
        pltpu.SemaphoreType.DMA,
        pltpu.SemaphoreType.DMA,
    ],
)


def _agg2_body(h1s, srcs, dsts, zeros, out, src_v, dst_v, rows0, rows1,
               table, g0, g1):
    cid = lax.axis_index("c")
    sid = lax.axis_index("s")
    pltpu.sync_copy(zeros, table.at[pl.ds(sid * ROWS_PER_TILE, ROWS_PER_TILE)])
    plsc.subcore_barrier()

    def run(half):
        _pipelined_segs(
            half,
            lambda g: srcs.at[sid, pl.ds(g * CPS_B, CPS_B)],
            lambda g: dsts.at[sid, pl.ds(g * CPS_B, CPS_B)],
            NCHUNK_B2 // CPS_B, CPS_B, table, src_v, dst_v, rows0, rows1,
            g0, g1)

    @pl.when(cid == 0)
    def _():
        run(h1s.at[0])

    @pl.when(cid == 1)
    def _():
        run(h1s.at[1])

    plsc.subcore_barrier()
    sl = pl.ds(sid * ROWS_PER_TILE, ROWS_PER_TILE)
    pltpu.sync_copy(table.at[sl], out.at[cid, sl])


_agg2 = pl.kernel(
    _agg2_body,
    out_type=jax.ShapeDtypeStruct((NC, NPAD, C_IN), jnp.float32),
    mesh=_mesh,
    compiler_params=pltpu.CompilerParams(use_tc_tiling_on_sc=False),
    scratch_types=[
        pltpu.VMEM((CPS_B, CHUNK_B), jnp.int32),
        pltpu.VMEM((CPS_B, CHUNK_B), jnp.int32),
        pltpu.VMEM((CHUNK_B, C_IN), jnp.float32),
        pltpu.VMEM((CHUNK_B, C_IN), jnp.float32),
        pltpu.VMEM_SHARED((NPAD, C_IN), jnp.float32),
        pltpu.SemaphoreType.DMA,
        pltpu.SemaphoreType.DMA,
    ],
)


def _elu(h):
    return jnp.where(h > 0, h, jnp.exp(jnp.minimum(h, 0.0)) - 1.0)


def _tc1_body(agg_ref, x_ref, wl_ref, wr_ref, b_ref, h_ref, inv_ref):
    agg = agg_ref[...]
    cnt = agg[0, :, C_IN:C_IN + 1] + agg[1, :, C_IN:C_IN + 1]
    inv = 1.0 / jnp.maximum(cnt, 1.0)
    mean = (agg[0, :, :C_IN] + agg[1, :, :C_IN]) * inv
    h = (jnp.dot(mean, wl_ref[...], preferred_element_type=jnp.float32)
         + jnp.dot(x_ref[...], wr_ref[...], preferred_element_type=jnp.float32)
         + b_ref[...])
    h = _elu(h)
    h_ref[0] = h[:, :C_IN]
    h_ref[1] = h[:, C_IN:]
    inv_ref[...] = inv


def _tc1(agg1, x, wl, wr, b):
    return pl.pallas_call(
        _tc1_body,
        grid=(G,),
        in_specs=[
            pl.BlockSpec((NC, R, C_AUG), lambda i: (0, i, 0)),
            pl.BlockSpec((R, C_IN), lambda i: (i, 0)),
            pl.BlockSpec((C_IN, HID), lambda i: (0, 0)),
            pl.BlockSpec((C_IN, HID), lambda i: (0, 0)),
            pl.BlockSpec((1, HID), lambda i: (0, 0)),
        ],
        out_specs=[
            pl.BlockSpec((NC, R, C_IN), lambda i: (0, i, 0)),
            pl.BlockSpec((R, 1), lambda i: (i, 0)),
        ],
        out_shape=[
            jax.ShapeDtypeStruct((NC, N, C_IN), jnp.float32),
            jax.ShapeDtypeStruct((N, 1), jnp.float32),
        ],
    )(agg1, x, wl, wr, b)


def _tc2_body(agg_ref, h1_ref, inv_ref, w2l_ref, w2r_ref, b2_ref,
              wf1_ref, bf1_ref, wf2_ref, bf2_ref, out_ref):
    inv = inv_ref[...]
    agg = agg_ref[...]
    h1 = h1_ref[...]
    w2l = w2l_ref[...]
    w2r = w2r_ref[...]
    f32 = jnp.float32
    z = (jnp.dot(agg[0] * inv, w2l[:C_IN], preferred_element_type=f32)
         + jnp.dot(agg[1] * inv, w2l[C_IN:], preferred_element_type=f32)
         + jnp.dot(h1[0], w2r[:C_IN], preferred_element_type=f32)
         + jnp.dot(h1[1], w2r[C_IN:], preferred_element_type=f32)
         + b2_ref[...])
    z = _elu(z)
    u = jnp.maximum(jnp.dot(z, wf1_ref[...], preferred_element_type=f32)
                    + bf1_ref[...], 0.0)
    out_ref[...] = jnp.dot(u, wf2_ref[...], preferred_element_type=f32) + bf2_ref[...]


def _tc2(agg2, h1s, invc, w2l, w2r, b2, wf1, bf1, wf2, bf2):
    return pl.pallas_call(
        _tc2_body,
        grid=(G,),
        in_specs=[
            pl.BlockSpec((NC, R, C_IN), lambda i: (0, i, 0)),
            pl.BlockSpec((NC, R, C_IN), lambda i: (0, i, 0)),
            pl.BlockSpec((R, 1), lambda i: (i, 0)),
            pl.BlockSpec((HID, HID), lambda i: (0, 0)),
            pl.BlockSpec((HID, HID), lambda i: (0, 0)),
            pl.BlockSpec((1, HID), lambda i: (0, 0)),
            pl.BlockSpec((HID, HID // 2), lambda i: (0, 0)),
            pl.BlockSpec((1, HID // 2), lambda i: (0, 0)),
            pl.BlockSpec((HID // 2, 1), lambda i: (0, 0)),
            pl.BlockSpec((1, 1), lambda i: (0, 0)),
        ],
        out_specs=pl.BlockSpec((R, 1), lambda i: (i, 0)),
        out_shape=jax.ShapeDtypeStruct((N, 1), jnp.float32),
    )(agg2, h1s, invc, w2l, w2r, b2, wf1, bf1, wf2, bf2)


def kernel(x, edge_index, W1l, b1l, W1r, W2l, b2l, W2r, Wf1, bf1, Wf2, bf2):
    ei = edge_index.astype(jnp.int32)
    src, dst = ei[0], ei[1]
    src_a = src.reshape(NC, NS, NCHUNK_A, CHUNK)
    dst_a = dst.reshape(NC, NS, NCHUNK_A, CHUNK)
    xaug = jnp.concatenate(
        [x, jnp.ones((N, 1), x.dtype), jnp.zeros((N, C_AUG - C_IN - 1), x.dtype)],
        axis=1)
    zeros_a = jnp.zeros((ROWS_PER_TILE, C_AUG), jnp.float32)
    agg1 = _agg1(xaug, src_a, dst_a, zeros_a)

    h1s, invc = _tc1(agg1, x, W1l.T, W1r.T, b1l[None, :])

    pad_w = NS * NCHUNK_B2 * CHUNK_B // E
    src_b = jnp.pad(src.reshape(NS, E // NS), ((0, 0), (0, 480))
                    ).reshape(NS, NCHUNK_B2, CHUNK_B)
    dst_b = jnp.pad(dst.reshape(NS, E // NS), ((0, 0), (0, 480)),
                    constant_values=NPAD - 1).reshape(NS, NCHUNK_B2, CHUNK_B)
    zeros_b = jnp.zeros((ROWS_PER_TILE, C_IN), jnp.float32)
    agg2 = _agg2(h1s, src_b, dst_b, zeros_b)

    out = _tc2(agg2, h1s, invc, W2l.T, W2r.T, b2l[None, :],
               Wf1.T, bf1[None, :], Wf2.T, bf2[None, :])
    return out[:, 0]


# speedup vs baseline: 1.0860x; 1.0860x over previous
import jax
import jax.numpy as jnp
from jax import lax
from jax.experimental import pallas as pl
from jax.experimental.pallas import tpu as pltpu
from jax.experimental.pallas import tpu_sc as plsc

N = 10000
E = 320000
C_IN = 128
C_AUG = 144
HID = 256
NC, NS = 2, 16
NPAD = 10240
ROWS_PER_TILE = NPAD // NS
CHUNK = 100
NCHUNK_A = E // (NC * NS) // CHUNK
NCHUNK_B = E // NS // CHUNK
CPS_A = 20
CHUNK_B = 128
NCHUNK_B2 = 20480 // CHUNK_B
CPS_B = 40
R = 400
G = N // R

_mesh = plsc.VectorSubcoreMesh(core_axis_name="c", subcore_axis_name="s")


def _pipelined_segs(gather_src, srcs_slice, dsts_slice, nseg, cps, table,
                    src_v, dst_v, rows0, rows1, g0, g1, gather_cols=None):
    nrows = rows0.shape[0]

    def gwin(buf):
        if gather_cols is None:
            return buf
        return buf.at[pl.ds(0, nrows), pl.ds(0, gather_cols)]

    def seg(g, carry):
        pltpu.sync_copy(srcs_slice(g), src_v)
        pltpu.sync_copy(dsts_slice(g), dst_v)

        def chunk(j, c):
            pltpu.async_copy(gather_src.at[src_v.at[j]], gwin(rows0), g0)
            pltpu.make_async_copy(gather_src.at[src_v.at[j]], gwin(rows0), g0).wait()
            pltpu.sync_copy(rows0, table.at[dst_v.at[j]], add=True)
            return c

        return lax.fori_loop(0, cps, chunk, carry)

    lax.fori_loop(0, nseg, seg, 0)


def _pipe_segs(gather_src, srcs_slice, dsts_slice, nseg, cps, table,
               src_v, dst_v, rows0, rows1, g0, g1, gather_cols=None):
    nrows = rows0.shape[0]

    def gwin(buf):
        if gather_cols is None:
            return buf
        return buf.at[pl.ds(0, nrows), pl.ds(0, gather_cols)]

    def seg(g, carry):
        pltpu.sync_copy(srcs_slice(g), src_v)
        pltpu.sync_copy(dsts_slice(g), dst_v)
        pltpu.async_copy(gather_src.at[src_v.at[0]], gwin(rows0), g0)

        def pair(p, c):
            j = 2 * p
            pltpu.async_copy(gather_src.at[src_v.at[j + 1]], gwin(rows1), g1)
            pltpu.make_async_copy(gather_src.at[src_v.at[j]], gwin(rows0), g0).wait()
            pltpu.sync_copy(rows0, table.at[dst_v.at[j]], add=True)

            @pl.when(p < cps // 2 - 1)
            def _():
                pltpu.async_copy(gather_src.at[src_v.at[j + 2]], gwin(rows0), g0)

            pltpu.make_async_copy(gather_src.at[src_v.at[j + 1]], gwin(rows1), g1).wait()
            pltpu.sync_copy(rows1, table.at[dst_v.at[j + 1]], add=True)
            return c

        return lax.fori_loop(0, cps // 2, pair, carry)

    lax.fori_loop(0, nseg, seg, 0)


def _agg1_body(xaug, srcs, dsts, zeros, out, src_v, dst_v, rows0, rows1,
               table, g0, g1):
    cid = lax.axis_index("c")
    sid = lax.axis_index("s")
    pltpu.sync_copy(zeros, table.at[pl.ds(sid * ROWS_PER_TILE, ROWS_PER_TILE)])
    plsc.subcore_barrier()
    _pipe_segs(
        xaug,
        lambda g: srcs.at[cid, sid, pl.ds(g * CPS_A, CPS_A)],
        lambda g: dsts.at[cid, sid, pl.ds(g * CPS_A, CPS_A)],
        NCHUNK_A // CPS_A, CPS_A, table, src_v, dst_v, rows0, rows1, g0, g1)
    plsc.subcore_barrier()
    sl = pl.ds(sid * ROWS_PER_TILE, ROWS_PER_TILE)
    pltpu.sync_copy(table.at[sl], out.at[cid, sl])


_agg1 = pl.kernel(
    _agg1_body,
    out_type=jax.ShapeDtypeStruct((NC, NPAD, C_AUG), jnp.float32),
    mesh=_mesh,
    compiler_params=pltpu.CompilerParams(use_tc_tiling_on_sc=False),
    scratch_types=[
        pltpu.VMEM((CPS_A, CHUNK), jnp.int32),
        pltpu.VMEM((CPS_A, CHUNK), jnp.int32),
        pltpu.VMEM((CHUNK, C_AUG), jnp.float32),
        pltpu.VMEM((CHUNK, C_AUG), jnp.float32),
        pltpu.VMEM_SHARED((NPAD, C_AUG), jnp.float32),
        pltpu.SemaphoreType.DMA,
        pltpu.SemaphoreType.DMA,
    ],
)


def _agg2_body(h1s, srcs, dsts, zeros, out, src_v, dst_v, rows0, rows1,
               table, g0, g1):
    cid = lax.axis_index("c")
    sid = lax.axis_index("s")
    pltpu.sync_copy(zeros, table.at[pl.ds(sid * ROWS_PER_TILE, ROWS_PER_TILE)])
    plsc.subcore_barrier()

    def run(half):
        _pipelined_segs(
            half,
            lambda g: srcs.at[sid, pl.ds(g * CPS_B, CPS_B)],
            lambda g: dsts.at[sid, pl.ds(g * CPS_B, CPS_B)],
            NCHUNK_B2 // CPS_B, CPS_B, table, src_v, dst_v, rows0, rows1,
            g0, g1)

    @pl.when(cid == 0)
    def _():
        run(h1s.at[0])

    @pl.when(cid == 1)
    def _():
        run(h1s.at[1])

    plsc.subcore_barrier()
    sl = pl.ds(sid * ROWS_PER_TILE, ROWS_PER_TILE)
    pltpu.sync_copy(table.at[sl], out.at[cid, sl])


_agg2 = pl.kernel(
    _agg2_body,
    out_type=jax.ShapeDtypeStruct((NC, NPAD, C_IN), jnp.float32),
    mesh=_mesh,
    scratch_types=[
        pltpu.VMEM((CPS_B, CHUNK_B), jnp.int32),
        pltpu.VMEM((CPS_B, CHUNK_B), jnp.int32),
        pltpu.VMEM((CHUNK_B, C_IN), jnp.float32),
        pltpu.VMEM((CHUNK_B, C_IN), jnp.float32),
        pltpu.VMEM_SHARED((NPAD, C_IN), jnp.float32),
        pltpu.SemaphoreType.DMA,
        pltpu.SemaphoreType.DMA,
    ],
)


def _elu(h):
    return jnp.where(h > 0, h, jnp.exp(jnp.minimum(h, 0.0)) - 1.0)


def _tc1_body(agg_ref, x_ref, wl_ref, wr_ref, b_ref, h_ref, inv_ref):
    agg = agg_ref[...]
    cnt = agg[0, :, C_IN:C_IN + 1] + agg[1, :, C_IN:C_IN + 1]
    inv = 1.0 / jnp.maximum(cnt, 1.0)
    mean = (agg[0, :, :C_IN] + agg[1, :, :C_IN]) * inv
    h = (jnp.dot(mean, wl_ref[...], preferred_element_type=jnp.float32)
         + jnp.dot(x_ref[...], wr_ref[...], preferred_element_type=jnp.float32)
         + b_ref[...])
    h = _elu(h)
    h_ref[0] = h[:, :C_IN]
    h_ref[1] = h[:, C_IN:]
    inv_ref[...] = inv


def _tc1(agg1, x, wl, wr, b):
    return pl.pallas_call(
        _tc1_body,
        grid=(G,),
        in_specs=[
            pl.BlockSpec((NC, R, C_AUG), lambda i: (0, i, 0)),
            pl.BlockSpec((R, C_IN), lambda i: (i, 0)),
            pl.BlockSpec((C_IN, HID), lambda i: (0, 0)),
            pl.BlockSpec((C_IN, HID), lambda i: (0, 0)),
            pl.BlockSpec((1, HID), lambda i: (0, 0)),
        ],
        out_specs=[
            pl.BlockSpec((NC, R, C_IN), lambda i: (0, i, 0)),
            pl.BlockSpec((R, 1), lambda i: (i, 0)),
        ],
        out_shape=[
            jax.ShapeDtypeStruct((NC, N, C_IN), jnp.float32),
            jax.ShapeDtypeStruct((N, 1), jnp.float32),
        ],
    )(agg1, x, wl, wr, b)


def _tc2_body(agg_ref, h1_ref, inv_ref, w2l_ref, w2r_ref, b2_ref,
              wf1_ref, bf1_ref, wf2_ref, bf2_ref, out_ref):
    inv = inv_ref[...]
    agg = agg_ref[...]
    h1 = h1_ref[...]
    w2l = w2l_ref[...]
    w2r = w2r_ref[...]
    f32 = jnp.float32
    z = (jnp.dot(agg[0] * inv, w2l[:C_IN], preferred_element_type=f32)
         + jnp.dot(agg[1] * inv, w2l[C_IN:], preferred_element_type=f32)
         + jnp.dot(h1[0], w2r[:C_IN], preferred_element_type=f32)
         + jnp.dot(h1[1], w2r[C_IN:], preferred_element_type=f32)
         + b2_ref[...])
    z = _elu(z)
    u = jnp.maximum(jnp.dot(z, wf1_ref[...], preferred_element_type=f32)
                    + bf1_ref[...], 0.0)
    out_ref[...] = jnp.dot(u, wf2_ref[...], preferred_element_type=f32) + bf2_ref[...]


def _tc2(agg2, h1s, invc, w2l, w2r, b2, wf1, bf1, wf2, bf2):
    return pl.pallas_call(
        _tc2_body,
        grid=(G,),
        in_specs=[
            pl.BlockSpec((NC, R, C_IN), lambda i: (0, i, 0)),
            pl.BlockSpec((NC, R, C_IN), lambda i: (0, i, 0)),
            pl.BlockSpec((R, 1), lambda i: (i, 0)),
            pl.BlockSpec((HID, HID), lambda i: (0, 0)),
            pl.BlockSpec((HID, HID), lambda i: (0, 0)),
            pl.BlockSpec((1, HID), lambda i: (0, 0)),
            pl.BlockSpec((HID, HID // 2), lambda i: (0, 0)),
            pl.BlockSpec((1, HID // 2), lambda i: (0, 0)),
            pl.BlockSpec((HID // 2, 1), lambda i: (0, 0)),
            pl.BlockSpec((1, 1), lambda i: (0, 0)),
        ],
        out_specs=pl.BlockSpec((R, 1), lambda i: (i, 0)),
        out_shape=jax.ShapeDtypeStruct((N, 1), jnp.float32),
    )(agg2, h1s, invc, w2l, w2r, b2, wf1, bf1, wf2, bf2)


def kernel(x, edge_index, W1l, b1l, W1r, W2l, b2l, W2r, Wf1, bf1, Wf2, bf2):
    ei = edge_index.astype(jnp.int32)
    src, dst = ei[0], ei[1]
    src_a = src.reshape(NC, NS, NCHUNK_A, CHUNK)
    dst_a = dst.reshape(NC, NS, NCHUNK_A, CHUNK)
    xaug = jnp.concatenate(
        [x, jnp.ones((N, 1), x.dtype), jnp.zeros((N, C_AUG - C_IN - 1), x.dtype)],
        axis=1)
    zeros_a = jnp.zeros((ROWS_PER_TILE, C_AUG), jnp.float32)
    agg1 = _agg1(xaug, src_a, dst_a, zeros_a)

    h1s, invc = _tc1(agg1, x, W1l.T, W1r.T, b1l[None, :])

    pad_w = NS * NCHUNK_B2 * CHUNK_B // E
    src_b = jnp.pad(src.reshape(NS, E // NS), ((0, 0), (0, 480))
                    ).reshape(NS, NCHUNK_B2, CHUNK_B)
    dst_b = jnp.pad(dst.reshape(NS, E // NS), ((0, 0), (0, 480)),
                    constant_values=NPAD - 1).reshape(NS, NCHUNK_B2, CHUNK_B)
    zeros_b = jnp.zeros((ROWS_PER_TILE, C_IN), jnp.float32)
    agg2 = _agg2(h1s, src_b, dst_b, zeros_b)

    out = _tc2(agg2, h1s, invc, W2l.T, W2r.T, b2l[None, :],
               Wf1.T, bf1[None, :], Wf2.T, bf2[None, :])
    return out[:, 0]


# speedup vs baseline: 1.9538x; 1.7990x over previous
import jax
import jax.numpy as jnp
from jax import lax
from jax.experimental import pallas as pl
from jax.experimental.pallas import tpu as pltpu
from jax.experimental.pallas import tpu_sc as plsc

N = 10000
E = 320000
C_IN = 128
C_AUG = 144
HID = 256
NC, NS = 2, 16
NPAD = 10240
ROWS_PER_TILE = NPAD // NS
CHUNK = 100
NCHUNK_A = E // (NC * NS) // CHUNK
NCHUNK_B = E // NS // CHUNK
CPS_A = 20
CHUNK_B = 100
NCHUNK_B2 = E // NS // CHUNK_B
CPS_B = 40
R = 400
G = N // R

_mesh = plsc.VectorSubcoreMesh(core_axis_name="c", subcore_axis_name="s")


def _pipelined_segs(gather_src, srcs_slice, dsts_slice, nseg, cps, table,
                    src_v, dst_v, rows0, rows1, g0, g1, gather_cols=None):
    nrows = rows0.shape[0]

    def gwin(buf):
        if gather_cols is None:
            return buf
        return buf.at[pl.ds(0, nrows), pl.ds(0, gather_cols)]

    def seg(g, carry):
        pltpu.sync_copy(srcs_slice(g), src_v)
        pltpu.sync_copy(dsts_slice(g), dst_v)

        def chunk(j, c):
            pltpu.async_copy(gather_src.at[src_v.at[j]], gwin(rows0), g0)
            pltpu.make_async_copy(gather_src.at[src_v.at[j]], gwin(rows0), g0).wait()
            pltpu.sync_copy(rows0, table.at[dst_v.at[j]], add=True)
            return c

        return lax.fori_loop(0, cps, chunk, carry)

    lax.fori_loop(0, nseg, seg, 0)


def _pipe_segs(gather_src, srcs_slice, dsts_slice, nseg, cps, table,
               src_v, dst_v, rows0, rows1, g0, g1, gather_cols=None):
    nrows = rows0.shape[0]

    def gwin(buf):
        if gather_cols is None:
            return buf
        return buf.at[pl.ds(0, nrows), pl.ds(0, gather_cols)]

    def seg(g, carry):
        pltpu.sync_copy(srcs_slice(g), src_v)
        pltpu.sync_copy(dsts_slice(g), dst_v)
        pltpu.async_copy(gather_src.at[src_v.at[0]], gwin(rows0), g0)

        def pair(p, c):
            j = 2 * p
            pltpu.async_copy(gather_src.at[src_v.at[j + 1]], gwin(rows1), g1)
            pltpu.make_async_copy(gather_src.at[src_v.at[j]], gwin(rows0), g0).wait()
            pltpu.sync_copy(rows0, table.at[dst_v.at[j]], add=True)

            @pl.when(p < cps // 2 - 1)
            def _():
                pltpu.async_copy(gather_src.at[src_v.at[j + 2]], gwin(rows0), g0)

            pltpu.make_async_copy(gather_src.at[src_v.at[j + 1]], gwin(rows1), g1).wait()
            pltpu.sync_copy(rows1, table.at[dst_v.at[j + 1]], add=True)
            return c

        return lax.fori_loop(0, cps // 2, pair, carry)

    lax.fori_loop(0, nseg, seg, 0)


def _agg1_body(xaug, srcs, dsts, zeros, out, src_v, dst_v, rows0, rows1,
               table, g0, g1):
    cid = lax.axis_index("c")
    sid = lax.axis_index("s")
    pltpu.sync_copy(zeros, table.at[pl.ds(sid * ROWS_PER_TILE, ROWS_PER_TILE)])
    plsc.subcore_barrier()
    _pipe_segs(
        xaug,
        lambda g: srcs.at[cid, sid, pl.ds(g * CPS_A, CPS_A)],
        lambda g: dsts.at[cid, sid, pl.ds(g * CPS_A, CPS_A)],
        NCHUNK_A // CPS_A, CPS_A, table, src_v, dst_v, rows0, rows1, g0, g1)
    plsc.subcore_barrier()
    sl = pl.ds(sid * ROWS_PER_TILE, ROWS_PER_TILE)
    pltpu.sync_copy(table.at[sl], out.at[cid, sl])


_agg1 = pl.kernel(
    _agg1_body,
    out_type=jax.ShapeDtypeStruct((NC, NPAD, C_AUG), jnp.float32),
    mesh=_mesh,
    compiler_params=pltpu.CompilerParams(use_tc_tiling_on_sc=False),
    scratch_types=[
        pltpu.VMEM((CPS_A, CHUNK), jnp.int32),
        pltpu.VMEM((CPS_A, CHUNK), jnp.int32),
        pltpu.VMEM((CHUNK, C_AUG), jnp.float32),
        pltpu.VMEM((CHUNK, C_AUG), jnp.float32),
        pltpu.VMEM_SHARED((NPAD, C_AUG), jnp.float32),
        pltpu.SemaphoreType.DMA,
        pltpu.SemaphoreType.DMA,
    ],
)


def _agg2_body(h1s, srcs, dsts, zeros, out, src_v, dst_v, rows0, rows1,
               table, g0, g1):
    cid = lax.axis_index("c")
    sid = lax.axis_index("s")
    pltpu.sync_copy(zeros, table.at[pl.ds(sid * ROWS_PER_TILE, ROWS_PER_TILE)])
    plsc.subcore_barrier()

    def run(half):
        _pipe_segs(
            half,
            lambda g: srcs.at[sid, pl.ds(g * CPS_B, CPS_B)],
            lambda g: dsts.at[sid, pl.ds(g * CPS_B, CPS_B)],
            NCHUNK_B2 // CPS_B, CPS_B, table, src_v, dst_v, rows0, rows1,
            g0, g1)

    @pl.when(cid == 0)
    def _():
        run(h1s.at[0])

    @pl.when(cid == 1)
    def _():
        run(h1s.at[1])

    plsc.subcore_barrier()
    sl = pl.ds(sid * ROWS_PER_TILE, ROWS_PER_TILE)
    pltpu.sync_copy(table.at[sl], out.at[cid, sl])


_agg2 = pl.kernel(
    _agg2_body,
    out_type=jax.ShapeDtypeStruct((NC, NPAD, C_IN), jnp.float32),
    mesh=_mesh,
    scratch_types=[
        pltpu.VMEM((CPS_B, CHUNK_B), jnp.int32),
        pltpu.VMEM((CPS_B, CHUNK_B), jnp.int32),
        pltpu.VMEM((CHUNK_B, C_IN), jnp.float32),
        pltpu.VMEM((CHUNK_B, C_IN), jnp.float32),
        pltpu.VMEM_SHARED((NPAD, C_IN), jnp.float32),
        pltpu.SemaphoreType.DMA,
        pltpu.SemaphoreType.DMA,
    ],
)


def _elu(h):
    return jnp.where(h > 0, h, jnp.exp(jnp.minimum(h, 0.0)) - 1.0)


def _tc1_body(agg_ref, x_ref, wl_ref, wr_ref, b_ref, h_ref, inv_ref):
    agg = agg_ref[...]
    cnt = agg[0, :, C_IN:C_IN + 1] + agg[1, :, C_IN:C_IN + 1]
    inv = 1.0 / jnp.maximum(cnt, 1.0)
    mean = (agg[0, :, :C_IN] + agg[1, :, :C_IN]) * inv
    h = (jnp.dot(mean, wl_ref[...], preferred_element_type=jnp.float32)
         + jnp.dot(x_ref[...], wr_ref[...], preferred_element_type=jnp.float32)
         + b_ref[...])
    h = _elu(h)
    h_ref[0] = h[:, :C_IN]
    h_ref[1] = h[:, C_IN:]
    inv_ref[...] = inv


def _tc1(agg1, x, wl, wr, b):
    return pl.pallas_call(
        _tc1_body,
        grid=(G,),
        in_specs=[
            pl.BlockSpec((NC, R, C_AUG), lambda i: (0, i, 0)),
            pl.BlockSpec((R, C_IN), lambda i: (i, 0)),
            pl.BlockSpec((C_IN, HID), lambda i: (0, 0)),
            pl.BlockSpec((C_IN, HID), lambda i: (0, 0)),
            pl.BlockSpec((1, HID), lambda i: (0, 0)),
        ],
        out_specs=[
            pl.BlockSpec((NC, R, C_IN), lambda i: (0, i, 0)),
            pl.BlockSpec((R, 1), lambda i: (i, 0)),
        ],
        out_shape=[
            jax.ShapeDtypeStruct((NC, N, C_IN), jnp.float32),
            jax.ShapeDtypeStruct((N, 1), jnp.float32),
        ],
    )(agg1, x, wl, wr, b)


def _tc2_body(agg_ref, h1_ref, inv_ref, w2l_ref, w2r_ref, b2_ref,
              wf1_ref, bf1_ref, wf2_ref, bf2_ref, out_ref):
    inv = inv_ref[...]
    agg = agg_ref[...]
    h1 = h1_ref[...]
    w2l = w2l_ref[...]
    w2r = w2r_ref[...]
    f32 = jnp.float32
    z = (jnp.dot(agg[0] * inv, w2l[:C_IN], preferred_element_type=f32)
         + jnp.dot(agg[1] * inv, w2l[C_IN:], preferred_element_type=f32)
         + jnp.dot(h1[0], w2r[:C_IN], preferred_element_type=f32)
         + jnp.dot(h1[1], w2r[C_IN:], preferred_element_type=f32)
         + b2_ref[...])
    z = _elu(z)
    u = jnp.maximum(jnp.dot(z, wf1_ref[...], preferred_element_type=f32)
                    + bf1_ref[...], 0.0)
    out_ref[...] = jnp.dot(u, wf2_ref[...], preferred_element_type=f32) + bf2_ref[...]


def _tc2(agg2, h1s, invc, w2l, w2r, b2, wf1, bf1, wf2, bf2):
    return pl.pallas_call(
        _tc2_body,
        grid=(G,),
        in_specs=[
            pl.BlockSpec((NC, R, C_IN), lambda i: (0, i, 0)),
            pl.BlockSpec((NC, R, C_IN), lambda i: (0, i, 0)),
            pl.BlockSpec((R, 1), lambda i: (i, 0)),
            pl.BlockSpec((HID, HID), lambda i: (0, 0)),
            pl.BlockSpec((HID, HID), lambda i: (0, 0)),
            pl.BlockSpec((1, HID), lambda i: (0, 0)),
            pl.BlockSpec((HID, HID // 2), lambda i: (0, 0)),
            pl.BlockSpec((1, HID // 2), lambda i: (0, 0)),
            pl.BlockSpec((HID // 2, 1), lambda i: (0, 0)),
            pl.BlockSpec((1, 1), lambda i: (0, 0)),
        ],
        out_specs=pl.BlockSpec((R, 1), lambda i: (i, 0)),
        out_shape=jax.ShapeDtypeStruct((N, 1), jnp.float32),
    )(agg2, h1s, invc, w2l, w2r, b2, wf1, bf1, wf2, bf2)


def kernel(x, edge_index, W1l, b1l, W1r, W2l, b2l, W2r, Wf1, bf1, Wf2, bf2):
    ei = edge_index.astype(jnp.int32)
    src, dst = ei[0], ei[1]
    src_a = src.reshape(NC, NS, NCHUNK_A, CHUNK)
    dst_a = dst.reshape(NC, NS, NCHUNK_A, CHUNK)
    xaug = jnp.concatenate(
        [x, jnp.ones((N, 1), x.dtype), jnp.zeros((N, C_AUG - C_IN - 1), x.dtype)],
        axis=1)
    zeros_a = jnp.zeros((ROWS_PER_TILE, C_AUG), jnp.float32)
    agg1 = _agg1(xaug, src_a, dst_a, zeros_a)

    h1s, invc = _tc1(agg1, x, W1l.T, W1r.T, b1l[None, :])

    src_b = src.reshape(NS, NCHUNK_B2, CHUNK_B)
    dst_b = dst.reshape(NS, NCHUNK_B2, CHUNK_B)
    zeros_b = jnp.zeros((ROWS_PER_TILE, C_IN), jnp.float32)
    agg2 = _agg2(h1s, src_b, dst_b, zeros_b)

    out = _tc2(agg2, h1s, invc, W2l.T, W2r.T, b2l[None, :],
               Wf1.T, bf1[None, :], Wf2.T, bf2[None, :])
    return out[:, 0]


# speedup vs baseline: 2.1232x; 1.0867x over previous
import jax
import jax.numpy as jnp
from jax import lax
from jax.experimental import pallas as pl
from jax.experimental.pallas import tpu as pltpu
from jax.experimental.pallas import tpu_sc as plsc

N = 10000
E = 320000
C_IN = 128
C_AUG = 144
HID = 256
NC, NS = 2, 16
NPAD = 10240
ROWS_PER_TILE = NPAD // NS
CHUNK = 100
NCHUNK_A = E // (NC * NS) // CHUNK
NCHUNK_B = E // NS // CHUNK
CPS_A = 20
CHUNK_B = 100
NCHUNK_B2 = E // NS // CHUNK_B
CPS_B = 40
R = 400
G = N // R

_mesh = plsc.VectorSubcoreMesh(core_axis_name="c", subcore_axis_name="s")


def _pipelined_segs(gather_src, srcs_slice, dsts_slice, nseg, cps, table,
                    src_v, dst_v, rows0, rows1, g0, g1, gather_cols=None):
    nrows = rows0.shape[0]

    def gwin(buf):
        if gather_cols is None:
            return buf
        return buf.at[pl.ds(0, nrows), pl.ds(0, gather_cols)]

    def seg(g, carry):
        pltpu.sync_copy(srcs_slice(g), src_v)
        pltpu.sync_copy(dsts_slice(g), dst_v)

        def chunk(j, c):
            pltpu.async_copy(gather_src.at[src_v.at[j]], gwin(rows0), g0)
            pltpu.make_async_copy(gather_src.at[src_v.at[j]], gwin(rows0), g0).wait()
            pltpu.sync_copy(rows0, table.at[dst_v.at[j]], add=True)
            return c

        return lax.fori_loop(0, cps, chunk, carry)

    lax.fori_loop(0, nseg, seg, 0)


def _pipe_segs(gather_src, srcs_slice, dsts_slice, nseg, cps, table,
               src_v, dst_v, rows0, rows1, g0, g1, gather_cols=None):
    nrows = rows0.shape[0]

    def gwin(buf):
        if gather_cols is None:
            return buf
        return buf.at[pl.ds(0, nrows), pl.ds(0, gather_cols)]

    def seg(g, carry):
        pltpu.sync_copy(srcs_slice(g), src_v)
        pltpu.sync_copy(dsts_slice(g), dst_v)
        pltpu.async_copy(gather_src.at[src_v.at[0]], gwin(rows0), g0)

        def pair(p, c):
            j = 2 * p
            pltpu.async_copy(gather_src.at[src_v.at[j + 1]], gwin(rows1), g1)
            pltpu.make_async_copy(gather_src.at[src_v.at[j]], gwin(rows0), g0).wait()
            pltpu.sync_copy(rows0, table.at[dst_v.at[j]], add=True)

            @pl.when(p < cps // 2 - 1)
            def _():
                pltpu.async_copy(gather_src.at[src_v.at[j + 2]], gwin(rows0), g0)

            pltpu.make_async_copy(gather_src.at[src_v.at[j + 1]], gwin(rows1), g1).wait()
            pltpu.sync_copy(rows1, table.at[dst_v.at[j + 1]], add=True)
            return c

        return lax.fori_loop(0, cps // 2, pair, carry)

    lax.fori_loop(0, nseg, seg, 0)


CW = 16


def _agg1_body(x, srcs, dsts, zf, zc, cinit, outf, outc, src_v, dst_v,
               rows0, rows1, crow, tablef, tablec, g0, g1):
    cid = lax.axis_index("c")
    sid = lax.axis_index("s")
    sl = pl.ds(sid * ROWS_PER_TILE, ROWS_PER_TILE)
    pltpu.sync_copy(zf, tablef.at[sl])
    pltpu.sync_copy(zc, tablec.at[sl])
    pltpu.sync_copy(cinit, crow)
    plsc.subcore_barrier()

    def seg(g, carry):
        pltpu.sync_copy(srcs.at[cid, sid, pl.ds(g * CPS_A, CPS_A)], src_v)
        pltpu.sync_copy(dsts.at[cid, sid, pl.ds(g * CPS_A, CPS_A)], dst_v)
        pltpu.async_copy(x.at[src_v.at[0]], rows0, g0)

        def pair(p, c):
            j = 2 * p
            pltpu.async_copy(x.at[src_v.at[j + 1]], rows1, g1)
            pltpu.make_async_copy(x.at[src_v.at[j]], rows0, g0).wait()
            pltpu.sync_copy(rows0, tablef.at[dst_v.at[j]], add=True)
            pltpu.sync_copy(crow, tablec.at[dst_v.at[j]], add=True)

            @pl.when(p < CPS_A // 2 - 1)
            def _():
                pltpu.async_copy(x.at[src_v.at[j + 2]], rows0, g0)

            pltpu.make_async_copy(x.at[src_v.at[j + 1]], rows1, g1).wait()
            pltpu.sync_copy(rows1, tablef.at[dst_v.at[j + 1]], add=True)
            pltpu.sync_copy(crow, tablec.at[dst_v.at[j + 1]], add=True)
            return c

        return lax.fori_loop(0, CPS_A // 2, pair, carry)

    lax.fori_loop(0, NCHUNK_A // CPS_A, seg, 0)
    plsc.subcore_barrier()
    pltpu.sync_copy(tablef.at[sl], outf.at[cid, sl])
    pltpu.sync_copy(tablec.at[sl], outc.at[cid, sl])


_agg1 = pl.kernel(
    _agg1_body,
    out_type=[
        jax.ShapeDtypeStruct((NC, NPAD, C_IN), jnp.float32),
        jax.ShapeDtypeStruct((NC, NPAD, CW), jnp.float32),
    ],
    mesh=_mesh,
    compiler_params=pltpu.CompilerParams(use_tc_tiling_on_sc=False),
    scratch_types=[
        pltpu.VMEM((CPS_A, CHUNK), jnp.int32),
        pltpu.VMEM((CPS_A, CHUNK), jnp.int32),
        pltpu.VMEM((CHUNK, C_IN), jnp.float32),
        pltpu.VMEM((CHUNK, C_IN), jnp.float32),
        pltpu.VMEM((CHUNK, CW), jnp.float32),
        pltpu.VMEM_SHARED((NPAD, C_IN), jnp.float32),
        pltpu.VMEM_SHARED((NPAD, CW), jnp.float32),
        pltpu.SemaphoreType.DMA,
        pltpu.SemaphoreType.DMA,
    ],
)


def _agg2_body(h1s, srcs, dsts, zeros, out, src_v, dst_v, rows0, rows1,
               table, g0, g1):
    cid = lax.axis_index("c")
    sid = lax.axis_index("s")
    pltpu.sync_copy(zeros, table.at[pl.ds(sid * ROWS_PER_TILE, ROWS_PER_TILE)])
    plsc.subcore_barrier()

    def run(half):
        _pipe_segs(
            half,
            lambda g: srcs.at[sid, pl.ds(g * CPS_B, CPS_B)],
            lambda g: dsts.at[sid, pl.ds(g * CPS_B, CPS_B)],
            NCHUNK_B2 // CPS_B, CPS_B, table, src_v, dst_v, rows0, rows1,
            g0, g1)

    @pl.when(cid == 0)
    def _():
        run(h1s.at[0])

    @pl.when(cid == 1)
    def _():
        run(h1s.at[1])

    plsc.subcore_barrier()
    sl = pl.ds(sid * ROWS_PER_TILE, ROWS_PER_TILE)
    pltpu.sync_copy(table.at[sl], out.at[cid, sl])


_agg2 = pl.kernel(
    _agg2_body,
    out_type=jax.ShapeDtypeStruct((NC, NPAD, C_IN), jnp.float32),
    mesh=_mesh,
    scratch_types=[
        pltpu.VMEM((CPS_B, CHUNK_B), jnp.int32),
        pltpu.VMEM((CPS_B, CHUNK_B), jnp.int32),
        pltpu.VMEM((CHUNK_B, C_IN), jnp.float32),
        pltpu.VMEM((CHUNK_B, C_IN), jnp.float32),
        pltpu.VMEM_SHARED((NPAD, C_IN), jnp.float32),
        pltpu.SemaphoreType.DMA,
        pltpu.SemaphoreType.DMA,
    ],
)


def _elu(h):
    return jnp.where(h > 0, h, jnp.exp(jnp.minimum(h, 0.0)) - 1.0)


def _tc1_body(agg_ref, cnt_ref, x_ref, wl_ref, wr_ref, b_ref, h_ref, inv_ref):
    agg = agg_ref[...]
    cnt = cnt_ref[0, :, 0:1] + cnt_ref[1, :, 0:1]
    inv = 1.0 / jnp.maximum(cnt, 1.0)
    mean = (agg[0] + agg[1]) * inv
    h = (jnp.dot(mean, wl_ref[...], preferred_element_type=jnp.float32)
         + jnp.dot(x_ref[...], wr_ref[...], preferred_element_type=jnp.float32)
         + b_ref[...])
    h = _elu(h)
    h_ref[0] = h[:, :C_IN]
    h_ref[1] = h[:, C_IN:]
    inv_ref[...] = inv


def _tc1(aggf, aggc, x, wl, wr, b):
    return pl.pallas_call(
        _tc1_body,
        grid=(G,),
        in_specs=[
            pl.BlockSpec((NC, R, C_IN), lambda i: (0, i, 0)),
            pl.BlockSpec((NC, R, CW), lambda i: (0, i, 0)),
            pl.BlockSpec((R, C_IN), lambda i: (i, 0)),
            pl.BlockSpec((C_IN, HID), lambda i: (0, 0)),
            pl.BlockSpec((C_IN, HID), lambda i: (0, 0)),
            pl.BlockSpec((1, HID), lambda i: (0, 0)),
        ],
        out_specs=[
            pl.BlockSpec((NC, R, C_IN), lambda i: (0, i, 0)),
            pl.BlockSpec((R, 1), lambda i: (i, 0)),
        ],
        out_shape=[
            jax.ShapeDtypeStruct((NC, N, C_IN), jnp.float32),
            jax.ShapeDtypeStruct((N, 1), jnp.float32),
        ],
    )(aggf, aggc, x, wl, wr, b)


def _tc2_body(agg_ref, h1_ref, inv_ref, w2l_ref, w2r_ref, b2_ref,
              wf1_ref, bf1_ref, wf2_ref, bf2_ref, out_ref):
    inv = inv_ref[...]
    agg = agg_ref[...]
    h1 = h1_ref[...]
    w2l = w2l_ref[...]
    w2r = w2r_ref[...]
    f32 = jnp.float32
    z = (jnp.dot(agg[0] * inv, w2l[:C_IN], preferred_element_type=f32)
         + jnp.dot(agg[1] * inv, w2l[C_IN:], preferred_element_type=f32)
         + jnp.dot(h1[0], w2r[:C_IN], preferred_element_type=f32)
         + jnp.dot(h1[1], w2r[C_IN:], preferred_element_type=f32)
         + b2_ref[...])
    z = _elu(z)
    u = jnp.maximum(jnp.dot(z, wf1_ref[...], preferred_element_type=f32)
                    + bf1_ref[...], 0.0)
    out_ref[...] = jnp.dot(u, wf2_ref[...], preferred_element_type=f32) + bf2_ref[...]


def _tc2(agg2, h1s, invc, w2l, w2r, b2, wf1, bf1, wf2, bf2):
    return pl.pallas_call(
        _tc2_body,
        grid=(G,),
        in_specs=[
            pl.BlockSpec((NC, R, C_IN), lambda i: (0, i, 0)),
            pl.BlockSpec((NC, R, C_IN), lambda i: (0, i, 0)),
            pl.BlockSpec((R, 1), lambda i: (i, 0)),
            pl.BlockSpec((HID, HID), lambda i: (0, 0)),
            pl.BlockSpec((HID, HID), lambda i: (0, 0)),
            pl.BlockSpec((1, HID), lambda i: (0, 0)),
            pl.BlockSpec((HID, HID // 2), lambda i: (0, 0)),
            pl.BlockSpec((1, HID // 2), lambda i: (0, 0)),
            pl.BlockSpec((HID // 2, 1), lambda i: (0, 0)),
            pl.BlockSpec((1, 1), lambda i: (0, 0)),
        ],
        out_specs=pl.BlockSpec((R, 1), lambda i: (i, 0)),
        out_shape=jax.ShapeDtypeStruct((N, 1), jnp.float32),
    )(agg2, h1s, invc, w2l, w2r, b2, wf1, bf1, wf2, bf2)


def kernel(x, edge_index, W1l, b1l, W1r, W2l, b2l, W2r, Wf1, bf1, Wf2, bf2):
    ei = edge_index.astype(jnp.int32)
    src, dst = ei[0], ei[1]
    src_a = src.reshape(NC, NS, NCHUNK_A, CHUNK)
    dst_a = dst.reshape(NC, NS, NCHUNK_A, CHUNK)
    zf = jnp.zeros((ROWS_PER_TILE, C_IN), jnp.float32)
    zc = jnp.zeros((ROWS_PER_TILE, CW), jnp.float32)
    cinit = jnp.zeros((CHUNK, CW), jnp.float32).at[:, 0].set(1.0)
    aggf, aggc = _agg1(x, src_a, dst_a, zf, zc, cinit)

    h1s, invc = _tc1(aggf, aggc, x, W1l.T, W1r.T, b1l[None, :])

    src_b = src.reshape(NS, NCHUNK_B2, CHUNK_B)
    dst_b = dst.reshape(NS, NCHUNK_B2, CHUNK_B)
    zeros_b = jnp.zeros((ROWS_PER_TILE, C_IN), jnp.float32)
    agg2 = _agg2(h1s, src_b, dst_b, zeros_b)

    out = _tc2(agg2, h1s, invc, W2l.T, W2r.T, b2l[None, :],
               Wf1.T, bf1[None, :], Wf2.T, bf2[None, :])
    return out[:, 0]
